# baseline (speedup 1.0000x reference)
import functools

import jax
import jax.numpy as jnp
from jax import lax
from jax.experimental import pallas as pl
from jax.experimental.pallas import tpu as pltpu
from jax.experimental.pallas import tpu_sc as plsc

S = 64
K = 256
L = 16


def _spmv_body(x_hbm, idx_hbm, vals_hbm, out_hbm,
               x_v, idx_v, vals_v, acc_v, sem):
    cp_x = pltpu.make_async_copy(x_hbm, x_v, sem)
    cp_i = pltpu.make_async_copy(idx_hbm, idx_v, sem)
    cp_v = pltpu.make_async_copy(vals_hbm, vals_v, sem)
    cp_x.start()
    cp_i.start()
    cp_v.start()

    zero = jnp.zeros((L,), jnp.float32)
    for j in range(S // L):
        acc_v[pl.ds(j * L, L)] = zero

    cp_x.wait()
    cp_i.wait()
    cp_v.wait()

    for i in range(K // L):
        r = idx_v[0, pl.ds(i * L, L)]
        c = idx_v[1, pl.ds(i * L, L)]
        v = vals_v[pl.ds(i * L, L)]
        g = plsc.load_gather(x_v, [c])
        plsc.addupdate_scatter(acc_v, [r], v * g)

    pltpu.sync_copy(acc_v, out_hbm)


@jax.jit
def _spmv(x, idx, vals):
    mesh = plsc.VectorSubcoreMesh(
        core_axis_name="c", subcore_axis_name="s",
        num_cores=1, num_subcores=1)
    return pl.kernel(
        _spmv_body,
        out_type=jax.ShapeDtypeStruct((S,), jnp.float32),
        mesh=mesh,
        scratch_types=[
            pltpu.VMEM((S,), jnp.float32),
            pltpu.VMEM((2, K), jnp.int32),
            pltpu.VMEM((K,), jnp.float32),
            pltpu.VMEM((S,), jnp.float32),
            pltpu.SemaphoreType.DMA,
        ],
        compiler_params=pltpu.CompilerParams(needs_layout_passes=False),
    )(x, idx, vals)


def kernel(x, indices, values):
    return _spmv(x, indices.astype(jnp.int32), values)
